# Initial kernel scaffold; baseline (speedup 1.0000x reference)
import functools

import jax
import jax.numpy as jnp
from jax import lax
from jax.experimental import pallas as pl
from jax.experimental.pallas import tpu as pltpu
from jax.experimental.pallas import tpu_sc as plsc

N = 10000
E = 320000
B = 64
D_NODE = 128
D_EDGE = 16
D_OUT = 128
L = 16

NC = 2
NS = 16
NW = NC * NS

EPW = E // NW
SUB_E = 2000
NSUB_E = EPW // SUB_E
NPAD = 10240
NPW = NPAD // NW
NVR = D_NODE // L

_F32Z = functools.partial(jnp.zeros, (L,), jnp.float32)


def _lower_bound(ids_ref, lo, hi, val):
    def cond(state):
        l, h = state
        return l < h

    def body(state):
        l, h = state
        mid = (l + h) // 2
        p = ids_ref[mid] < val
        return jnp.where(p, mid + 1, l), jnp.where(p, h, mid)

    l, _ = lax.while_loop(cond, body, (jnp.int32(lo), jnp.int32(hi)))
    return l


def _tree_add(vals):
    while len(vals) > 1:
        vals = [a + b for a, b in zip(vals[::2], vals[1::2])]
    return vals[0]


def _accum_edges(ebuf, eidb, acc_e):
    g0 = eidb[0]
    g1 = eidb[SUB_E - 1]

    def seg_body(g, start):
        end = _lower_bound(eidb, start, SUB_E, g + 1)
        U = 8
        nb = (end - start) // U

        def blk(j, acc):
            base = start + j * U
            return acc + _tree_add([ebuf[base + u] for u in range(U)])

        acc = lax.fori_loop(0, nb, blk, _F32Z())

        def rem(i, acc):
            return acc + ebuf[i]

        acc = lax.fori_loop(start + nb * U, end, rem, acc)
        acc_e[g] = acc_e[g] + acc
        return end

    lax.fori_loop(g0, g1 + 1, seg_body, jnp.int32(0))


def _accum_nodes(nbuf, nidb, acc_n):
    g0 = nidb[0]
    g1 = nidb[NPW - 1]

    def seg_body(g, start):
        end = _lower_bound(nidb, start, NPW, g + 1)

        def row(i, accs):
            return tuple(accs[j] + nbuf[i, pl.ds(L * j, L)] for j in range(NVR))

        accs = lax.fori_loop(start, end, row,
                             tuple(_F32Z() for _ in range(NVR)))
        for j in range(NVR):
            sl = pl.ds(L * j, L)
            acc_n[g, sl] = acc_n[g, sl] + accs[j]
        return end

    lax.fori_loop(g0, g1 + 1, seg_body, jnp.int32(0))


def _sc_body(edges_hbm, eids_hbm, nodes_hbm, nids_hbm, eout_hbm, nout_hbm,
             ebuf, eidb, nbuf, nidb, acc_e, acc_n):
    c = lax.axis_index("c")
    s = lax.axis_index("s")
    w = s * NC + c

    z = _F32Z()

    def zero_row(i, _):
        acc_e[i] = z
        for j in range(NVR):
            acc_n[i, pl.ds(L * j, L)] = z
        return 0

    lax.fori_loop(0, B, zero_row, 0)

    for k in range(NSUB_E):
        base = w * EPW + k * SUB_E
        pltpu.sync_copy(edges_hbm.at[pl.ds(base, SUB_E)], ebuf)
        pltpu.sync_copy(eids_hbm.at[pl.ds(base, SUB_E)], eidb)
        _accum_edges(ebuf, eidb, acc_e)

    nbase = w * NPW
    pltpu.sync_copy(nodes_hbm.at[pl.ds(nbase, NPW)], nbuf)
    pltpu.sync_copy(nids_hbm.at[pl.ds(nbase, NPW)], nidb)
    _accum_nodes(nbuf, nidb, acc_n)

    pltpu.sync_copy(acc_e, eout_hbm.at[w])
    pltpu.sync_copy(acc_n, nout_hbm.at[w])


_sc_seg = functools.partial(
    pl.kernel,
    out_type=[
        jax.ShapeDtypeStruct((NW, B, D_EDGE), jnp.float32),
        jax.ShapeDtypeStruct((NW, B, D_NODE), jnp.float32),
    ],
    mesh=plsc.VectorSubcoreMesh(core_axis_name="c", subcore_axis_name="s"),
    scratch_types=[
        pltpu.VMEM((SUB_E, D_EDGE), jnp.float32),
        pltpu.VMEM((SUB_E,), jnp.int32),
        pltpu.VMEM((NPW, D_NODE), jnp.float32),
        pltpu.VMEM((NPW,), jnp.int32),
        pltpu.VMEM((B, D_EDGE), jnp.float32),
        pltpu.VMEM((B, D_NODE), jnp.float32),
    ],
)(_sc_body)


def _tc_body(ep_ref, np_ref, g_ref, we_ref, wn_ref, wg_ref, b_ref, out_ref):
    eagg = jnp.sum(ep_ref[...], axis=0)
    nagg = jnp.sum(np_ref[...], axis=0)
    acc = jnp.dot(eagg, we_ref[...], preferred_element_type=jnp.float32)
    acc = acc + jnp.dot(nagg, wn_ref[...], preferred_element_type=jnp.float32)
    acc = acc + jnp.dot(g_ref[...], wg_ref[...],
                        preferred_element_type=jnp.float32)
    out_ref[...] = acc + b_ref[...]


def _tc_mlp(eparts, nparts, graph_globals, We, Wn, Wg, b2):
    return pl.pallas_call(
        _tc_body,
        out_shape=jax.ShapeDtypeStruct((B, D_OUT), jnp.float32),
    )(eparts, nparts, graph_globals, We, Wn, Wg, b2)


def kernel(nodes, edges, graph_globals, node_graph_ids, edge_graph_ids, W, b):
    eids = edge_graph_ids.astype(jnp.int32)
    nids = node_graph_ids.astype(jnp.int32)
    pad = NPAD - N
    nodes_p = jnp.concatenate(
        [nodes, jnp.zeros((pad, D_NODE), nodes.dtype)], axis=0)
    nids_p = jnp.concatenate(
        [nids, jnp.full((pad,), B - 1, jnp.int32)], axis=0)
    eparts, nparts = _sc_seg(edges, eids, nodes_p, nids_p)
    We = W[:D_EDGE]
    Wn = W[D_EDGE:D_EDGE + D_NODE]
    Wg = W[D_EDGE + D_NODE:]
    return _tc_mlp(eparts, nparts, graph_globals, We, Wn, Wg,
                   b.reshape(1, D_OUT))


# speedup vs baseline: 5.7320x; 5.7320x over previous
import functools

import jax
import jax.numpy as jnp
from jax import lax
from jax.experimental import pallas as pl
from jax.experimental.pallas import tpu as pltpu
from jax.experimental.pallas import tpu_sc as plsc

N = 10000
E = 320000
B = 64
D_NODE = 128
D_EDGE = 16
D_OUT = 128
L = 16

NC = 2
NS = 16
NW = NC * NS

EPW = E // NW
SUB_E = 2000
NSUB_E = EPW // SUB_E
NPAD = 10240
NPW = NPAD // NW
NVR = D_NODE // L

_F32Z = functools.partial(jnp.zeros, (L,), jnp.float32)


def _sload(ref, idx):
    return ref[pl.ds(idx, L)][0]


def _lower_bound(ids_ref, lo, hi, val, steps):
    l = jnp.int32(lo)
    h = jnp.int32(hi)
    for _ in range(steps):
        mid = (l + h) // 2
        v = _sload(ids_ref, mid)
        active = l < h
        less = v < val
        l = jnp.where(jnp.logical_and(active, less), mid + 1, l)
        h = jnp.where(jnp.logical_and(active, jnp.logical_not(less)), mid, h)
    return l


def _tree_add(vals):
    while len(vals) > 1:
        vals = [a + b for a, b in zip(vals[::2], vals[1::2])]
    return vals[0]


def _accum_edges(ebuf, eidb, acc_e):
    g0 = _sload(eidb, 0)
    g1 = _sload(eidb, SUB_E - 1)

    def seg_body(g, start):
        end = _lower_bound(eidb, start, SUB_E, g + 1, 11)
        U = 8
        nb = (end - start) // U

        def blk(j, acc):
            base = start + j * U
            return acc + _tree_add([ebuf[base + u] for u in range(U)])

        acc = lax.fori_loop(0, nb, blk, _F32Z())

        def rem(i, acc):
            return acc + ebuf[i]

        acc = lax.fori_loop(start + nb * U, end, rem, acc)
        acc_e[g] = acc_e[g] + acc
        return end

    lax.fori_loop(g0, g1 + 1, seg_body, jnp.int32(0))


def _accum_nodes(nbuf, nidb, acc_n):
    g0 = _sload(nidb, 0)
    g1 = _sload(nidb, NPW - 1)

    def seg_body(g, start):
        end = _lower_bound(nidb, start, NPW, g + 1, 9)

        def row(i, accs):
            return tuple(accs[j] + nbuf[i, pl.ds(L * j, L)] for j in range(NVR))

        accs = lax.fori_loop(start, end, row,
                             tuple(_F32Z() for _ in range(NVR)))
        for j in range(NVR):
            sl = pl.ds(L * j, L)
            acc_n[g, sl] = acc_n[g, sl] + accs[j]
        return end

    lax.fori_loop(g0, g1 + 1, seg_body, jnp.int32(0))


def _sc_body(edges_hbm, eids_hbm, nodes_hbm, nids_hbm, eout_hbm, nout_hbm,
             ebuf, eidb, nbuf, nidb, acc_e, acc_n):
    c = lax.axis_index("c")
    s = lax.axis_index("s")
    w = s * NC + c

    z = _F32Z()

    def zero_row(i, _):
        acc_e[i] = z
        for j in range(NVR):
            acc_n[i, pl.ds(L * j, L)] = z
        return 0

    lax.fori_loop(0, B, zero_row, 0)

    for k in range(NSUB_E):
        base = w * EPW + k * SUB_E
        pltpu.sync_copy(edges_hbm.at[pl.ds(base, SUB_E)], ebuf)
        pltpu.sync_copy(eids_hbm.at[pl.ds(base, SUB_E)],
                        eidb.at[pl.ds(0, SUB_E)])
        _accum_edges(ebuf, eidb, acc_e)

    nbase = w * NPW
    pltpu.sync_copy(nodes_hbm.at[pl.ds(nbase, NPW)], nbuf)
    pltpu.sync_copy(nids_hbm.at[pl.ds(nbase, NPW)], nidb.at[pl.ds(0, NPW)])
    _accum_nodes(nbuf, nidb, acc_n)

    pltpu.sync_copy(acc_e, eout_hbm.at[w])
    pltpu.sync_copy(acc_n, nout_hbm.at[w])


_sc_seg = functools.partial(
    pl.kernel,
    out_type=[
        jax.ShapeDtypeStruct((NW, B, D_EDGE), jnp.float32),
        jax.ShapeDtypeStruct((NW, B, D_NODE), jnp.float32),
    ],
    mesh=plsc.VectorSubcoreMesh(core_axis_name="c", subcore_axis_name="s"),
    scratch_types=[
        pltpu.VMEM((SUB_E, D_EDGE), jnp.float32),
        pltpu.VMEM((SUB_E + L,), jnp.int32),
        pltpu.VMEM((NPW, D_NODE), jnp.float32),
        pltpu.VMEM((NPW + L,), jnp.int32),
        pltpu.VMEM((B, D_EDGE), jnp.float32),
        pltpu.VMEM((B, D_NODE), jnp.float32),
    ],
    compiler_params=pltpu.CompilerParams(use_tc_tiling_on_sc=False),
)(_sc_body)


def _tc_body(ep_ref, np_ref, g_ref, we_ref, wn_ref, wg_ref, b_ref, out_ref):
    eagg = jnp.sum(ep_ref[...], axis=0)
    nagg = jnp.sum(np_ref[...], axis=0)
    acc = jnp.dot(eagg, we_ref[...], preferred_element_type=jnp.float32)
    acc = acc + jnp.dot(nagg, wn_ref[...], preferred_element_type=jnp.float32)
    acc = acc + jnp.dot(g_ref[...], wg_ref[...],
                        preferred_element_type=jnp.float32)
    out_ref[...] = acc + b_ref[...]


def _tc_mlp(eparts, nparts, graph_globals, We, Wn, Wg, b2):
    return pl.pallas_call(
        _tc_body,
        out_shape=jax.ShapeDtypeStruct((B, D_OUT), jnp.float32),
    )(eparts, nparts, graph_globals, We, Wn, Wg, b2)


def kernel(nodes, edges, graph_globals, node_graph_ids, edge_graph_ids, W, b):
    eids = edge_graph_ids.astype(jnp.int32)
    nids = node_graph_ids.astype(jnp.int32)
    pad = NPAD - N
    nodes_p = jnp.concatenate(
        [nodes, jnp.zeros((pad, D_NODE), nodes.dtype)], axis=0)
    nids_p = jnp.concatenate(
        [nids, jnp.full((pad,), B - 1, jnp.int32)], axis=0)
    eparts, nparts = _sc_seg(edges, eids, nodes_p, nids_p)
    We = W[:D_EDGE]
    Wn = W[D_EDGE:D_EDGE + D_NODE]
    Wg = W[D_EDGE + D_NODE:]
    return _tc_mlp(eparts, nparts, graph_globals, We, Wn, Wg,
                   b.reshape(1, D_OUT))


# speedup vs baseline: 5.7699x; 1.0066x over previous
import functools

import jax
import jax.numpy as jnp
from jax import lax
from jax.experimental import pallas as pl
from jax.experimental.pallas import tpu as pltpu
from jax.experimental.pallas import tpu_sc as plsc

N = 10000
E = 320000
B = 64
D_NODE = 128
D_EDGE = 16
D_OUT = 128
L = 16

NC = 2
NS = 16
NW = NC * NS

EPW = E // NW
SUB_E = 2000
NSUB_E = EPW // SUB_E
NPW = 312
NPW_LAST = N - (NW - 1) * NPW
NVR = D_NODE // L

_F32Z = functools.partial(jnp.zeros, (L,), jnp.float32)


def _sload(ref, idx):
    return ref[pl.ds(idx, L)][0]


def _lower_bound(ids_ref, lo, hi, val, steps):
    l = jnp.int32(lo)
    h = jnp.int32(hi)
    for _ in range(steps):
        mid = (l + h) // 2
        v = _sload(ids_ref, mid)
        active = l < h
        less = v < val
        l = jnp.where(jnp.logical_and(active, less), mid + 1, l)
        h = jnp.where(jnp.logical_and(active, jnp.logical_not(less)), mid, h)
    return l


def _tree_add(vals):
    while len(vals) > 1:
        vals = [a + b for a, b in zip(vals[::2], vals[1::2])]
    return vals[0]


def _accum_edges(ebuf, eidb, acc_e):
    g0 = _sload(eidb, 0)
    g1 = _sload(eidb, SUB_E - 1)

    def seg_body(g, start):
        end = _lower_bound(eidb, start, SUB_E, g + 1, 11)
        U = 8
        nb = (end - start) // U

        def blk(j, acc):
            base = start + j * U
            return acc + _tree_add([ebuf[base + u] for u in range(U)])

        acc = lax.fori_loop(0, nb, blk, _F32Z())

        def rem(i, acc):
            return acc + ebuf[i]

        acc = lax.fori_loop(start + nb * U, end, rem, acc)
        acc_e[g] = acc_e[g] + acc
        return end

    lax.fori_loop(g0, g1 + 1, seg_body, jnp.int32(0))


def _accum_nodes(nbuf, nidb, acc_n, rows):
    g0 = _sload(nidb, 0)
    g1 = _sload(nidb, rows - 1)

    def seg_body(g, start):
        end = _lower_bound(nidb, start, rows, g + 1, 9)

        def row(i, accs):
            return tuple(accs[j] + nbuf[i, pl.ds(L * j, L)] for j in range(NVR))

        accs = lax.fori_loop(start, end, row,
                             tuple(_F32Z() for _ in range(NVR)))
        for j in range(NVR):
            sl = pl.ds(L * j, L)
            acc_n[g, sl] = acc_n[g, sl] + accs[j]
        return end

    lax.fori_loop(g0, g1 + 1, seg_body, jnp.int32(0))


def _sc_body(edges_hbm, eids_hbm, nodes_hbm, nids_hbm, eout_hbm, nout_hbm,
             ebuf, eidb, nbuf, nidb, acc_e, acc_n):
    c = lax.axis_index("c")
    s = lax.axis_index("s")
    w = s * NC + c

    z = _F32Z()

    def zero_row(i, _):
        acc_e[i] = z
        for j in range(NVR):
            acc_n[i, pl.ds(L * j, L)] = z
        return 0

    lax.fori_loop(0, B, zero_row, 0)

    for k in range(NSUB_E):
        base = w * EPW + k * SUB_E
        pltpu.sync_copy(edges_hbm.at[pl.ds(base, SUB_E)], ebuf)
        pltpu.sync_copy(eids_hbm.at[pl.ds(base, SUB_E)],
                        eidb.at[pl.ds(0, SUB_E)])
        _accum_edges(ebuf, eidb, acc_e)

    nbase = w * NPW
    pltpu.sync_copy(nodes_hbm.at[pl.ds(nbase, NPW_LAST)], nbuf)
    pltpu.sync_copy(nids_hbm.at[pl.ds(nbase, NPW_LAST)],
                    nidb.at[pl.ds(0, NPW_LAST)])
    rows = jnp.where(w == NW - 1, NPW_LAST, NPW).astype(jnp.int32)
    _accum_nodes(nbuf, nidb, acc_n, rows)

    pltpu.sync_copy(acc_e, eout_hbm.at[w])
    pltpu.sync_copy(acc_n, nout_hbm.at[w])


_sc_seg = functools.partial(
    pl.kernel,
    out_type=[
        jax.ShapeDtypeStruct((NW, B, D_EDGE), jnp.float32),
        jax.ShapeDtypeStruct((NW, B, D_NODE), jnp.float32),
    ],
    mesh=plsc.VectorSubcoreMesh(core_axis_name="c", subcore_axis_name="s"),
    scratch_types=[
        pltpu.VMEM((SUB_E, D_EDGE), jnp.float32),
        pltpu.VMEM((SUB_E + L,), jnp.int32),
        pltpu.VMEM((NPW_LAST, D_NODE), jnp.float32),
        pltpu.VMEM((NPW_LAST + L,), jnp.int32),
        pltpu.VMEM((B, D_EDGE), jnp.float32),
        pltpu.VMEM((B, D_NODE), jnp.float32),
    ],
    compiler_params=pltpu.CompilerParams(use_tc_tiling_on_sc=False),
)(_sc_body)


def _tc_body(ep_ref, np_ref, g_ref, we_ref, wn_ref, wg_ref, b_ref, out_ref):
    eagg = jnp.sum(ep_ref[...], axis=0)
    nagg = jnp.sum(np_ref[...], axis=0)
    acc = jnp.dot(eagg, we_ref[...], preferred_element_type=jnp.float32)
    acc = acc + jnp.dot(nagg, wn_ref[...], preferred_element_type=jnp.float32)
    acc = acc + jnp.dot(g_ref[...], wg_ref[...],
                        preferred_element_type=jnp.float32)
    out_ref[...] = acc + b_ref[...]


def _tc_mlp(eparts, nparts, graph_globals, We, Wn, Wg, b2):
    return pl.pallas_call(
        _tc_body,
        out_shape=jax.ShapeDtypeStruct((B, D_OUT), jnp.float32),
    )(eparts, nparts, graph_globals, We, Wn, Wg, b2)


def kernel(nodes, edges, graph_globals, node_graph_ids, edge_graph_ids, W, b):
    eids = edge_graph_ids.astype(jnp.int32)
    nids = node_graph_ids.astype(jnp.int32)
    eparts, nparts = _sc_seg(edges, eids, nodes, nids)
    We = W[:D_EDGE]
    Wn = W[D_EDGE:D_EDGE + D_NODE]
    Wg = W[D_EDGE + D_NODE:]
    return _tc_mlp(eparts, nparts, graph_globals, We, Wn, Wg,
                   b.reshape(1, D_OUT))


# speedup vs baseline: 17.2616x; 2.9917x over previous
import functools

import jax
import jax.numpy as jnp
from jax import lax
from jax.experimental import pallas as pl
from jax.experimental.pallas import tpu as pltpu
from jax.experimental.pallas import tpu_sc as plsc

N = 10000
E = 320000
B = 64
D_NODE = 128
D_EDGE = 16
D_OUT = 128
L = 16

NC = 2
NS = 16
NW = NC * NS

NPW = 312
NPW_LAST = N - (NW - 1) * NPW
NVR = D_NODE // L

NT = E // 128
TPW = NT // NW
TREM = NT - TPW * NW
TSUB = 16
NSUB_E = 5
SUB_E = TSUB * 128

_F32Z = functools.partial(jnp.zeros, (L,), jnp.float32)


def _sload(ref, idx):
    return ref[pl.ds(idx, L)][0]


def _lower_bound(ids_ref, lo, hi, val, steps):
    l = jnp.int32(lo)
    h = jnp.int32(hi)
    for _ in range(steps):
        mid = (l + h) // 2
        v = _sload(ids_ref, mid)
        active = l < h
        less = v < val
        l = jnp.where(jnp.logical_and(active, less), mid + 1, l)
        h = jnp.where(jnp.logical_and(active, jnp.logical_not(less)), mid, h)
    return l


def _tree_add(vals):
    while len(vals) > 1:
        vals = [a + b for a, b in zip(vals[::2], vals[1::2])]
    return vals[0]


def _group_loads(ebuf, j):
    tj = j // 8
    cj = (j % 8) * L
    return [ebuf[f // 8, tj, f % 8, pl.ds(cj, L)] for f in range(16)]


def _accum_edges(ebuf, eidb, acc_e, start, end):
    g0 = _sload(eidb, start)
    g1 = _sload(eidb, end - 1)
    iot = lax.iota(jnp.int32, 16)
    z = _F32Z()

    def seg_body(g, s):
        e_ = _lower_bound(eidb, s, end, g + 1, 12)
        hg = s // L
        hs = s - hg * L
        he = jnp.minimum(e_ - hg * L, L)
        mh = jnp.logical_and(iot >= hs, iot < he)
        accs = tuple(jnp.where(mh, v, z) for v in _group_loads(ebuf, hg))

        def blk(j, accs):
            return tuple(a + v for a, v in zip(accs, _group_loads(ebuf, j)))

        tg = e_ // L
        accs = lax.fori_loop(hg + 1, tg, blk, accs)

        tlen = jnp.where(tg > hg, e_ - tg * L, 0)
        mt = iot < tlen
        accs = tuple(a + jnp.where(mt, v, z)
                     for a, v in zip(accs, _group_loads(ebuf, tg)))

        out_v = acc_e[g]
        for f in range(16):
            out_v = jnp.where(iot == f, out_v + jnp.sum(accs[f]), out_v)
        acc_e[g] = out_v
        return e_

    lax.fori_loop(g0, g1 + 1, seg_body, start)


def _accum_nodes(nbuf, nidb, acc_n, rows):
    g0 = _sload(nidb, 0)
    g1 = _sload(nidb, rows - 1)

    def seg_body(g, start):
        end = _lower_bound(nidb, start, rows, g + 1, 9)

        def row(i, accs):
            return tuple(accs[j] + nbuf[i, pl.ds(L * j, L)] for j in range(NVR))

        accs = lax.fori_loop(start, end, row,
                             tuple(_F32Z() for _ in range(NVR)))
        for j in range(NVR):
            sl = pl.ds(L * j, L)
            acc_n[g, sl] = acc_n[g, sl] + accs[j]
        return end

    lax.fori_loop(g0, g1 + 1, seg_body, jnp.int32(0))


def _sc_body(edges_hbm, eids_hbm, nodes_hbm, nids_hbm, eout_hbm, nout_hbm,
             ebuf, eidb, nbuf, nidb, acc_e, acc_n):
    c = lax.axis_index("c")
    s = lax.axis_index("s")
    w = s * NC + c

    z = _F32Z()

    def zero_row(i, _):
        acc_e[i] = z
        for j in range(NVR):
            acc_n[i, pl.ds(L * j, L)] = z
        return 0

    lax.fori_loop(0, B, zero_row, 0)

    tstart = TPW * w + jnp.minimum(w, TREM)
    tend = tstart + TPW + jnp.where(w < TREM, 1, 0)
    cur = tstart * 128
    for k in range(NSUB_E):
        tb = jnp.minimum(tstart + k * TSUB, NT - TSUB)
        for a in range(2):
            pltpu.sync_copy(edges_hbm.at[a, pl.ds(tb, TSUB)],
                            ebuf.at[a, pl.ds(0, TSUB)])
        pltpu.sync_copy(eids_hbm.at[pl.ds(tb * 128, SUB_E)],
                        eidb.at[pl.ds(0, SUB_E)])
        lo = jnp.maximum(cur, tb * 128)
        hi = jnp.minimum(tend * 128, tb * 128 + SUB_E)
        _accum_edges(ebuf, eidb, acc_e, lo - tb * 128, hi - tb * 128)
        cur = hi

    nbase = w * NPW
    pltpu.sync_copy(nodes_hbm.at[pl.ds(nbase, NPW_LAST)], nbuf)
    pltpu.sync_copy(nids_hbm.at[pl.ds(nbase, NPW_LAST)],
                    nidb.at[pl.ds(0, NPW_LAST)])
    rows = jnp.where(w == NW - 1, NPW_LAST, NPW).astype(jnp.int32)
    _accum_nodes(nbuf, nidb, acc_n, rows)

    pltpu.sync_copy(acc_e, eout_hbm.at[w])
    pltpu.sync_copy(acc_n, nout_hbm.at[w])


_sc_seg = functools.partial(
    pl.kernel,
    out_type=[
        jax.ShapeDtypeStruct((NW, B, D_EDGE), jnp.float32),
        jax.ShapeDtypeStruct((NW, B, D_NODE), jnp.float32),
    ],
    mesh=plsc.VectorSubcoreMesh(core_axis_name="c", subcore_axis_name="s"),
    scratch_types=[
        pltpu.VMEM((2, TSUB + 1, 8, 128), jnp.float32),
        pltpu.VMEM((SUB_E + L,), jnp.int32),
        pltpu.VMEM((NPW_LAST, D_NODE), jnp.float32),
        pltpu.VMEM((NPW_LAST + L,), jnp.int32),
        pltpu.VMEM((B, D_EDGE), jnp.float32),
        pltpu.VMEM((B, D_NODE), jnp.float32),
    ],
    compiler_params=pltpu.CompilerParams(use_tc_tiling_on_sc=False,
                                         needs_layout_passes=False),
)(_sc_body)


def _tc_body(ep_ref, np_ref, g_ref, we_ref, wn_ref, wg_ref, b_ref, out_ref):
    eagg = jnp.sum(ep_ref[...], axis=0)
    nagg = jnp.sum(np_ref[...], axis=0)
    acc = jnp.dot(eagg, we_ref[...], preferred_element_type=jnp.float32)
    acc = acc + jnp.dot(nagg, wn_ref[...], preferred_element_type=jnp.float32)
    acc = acc + jnp.dot(g_ref[...], wg_ref[...],
                        preferred_element_type=jnp.float32)
    out_ref[...] = acc + b_ref[...]


def _tc_mlp(eparts, nparts, graph_globals, We, Wn, Wg, b2):
    return pl.pallas_call(
        _tc_body,
        out_shape=jax.ShapeDtypeStruct((B, D_OUT), jnp.float32),
    )(eparts, nparts, graph_globals, We, Wn, Wg, b2)


def kernel(nodes, edges, graph_globals, node_graph_ids, edge_graph_ids, W, b):
    eids = edge_graph_ids.astype(jnp.int32)
    nids = node_graph_ids.astype(jnp.int32)
    edges4 = edges.T.reshape(2, 8, NT, 128).transpose(0, 2, 1, 3)
    eparts, nparts = _sc_seg(edges4, eids, nodes, nids)
    We = W[:D_EDGE]
    Wn = W[D_EDGE:D_EDGE + D_NODE]
    Wg = W[D_EDGE + D_NODE:]
    return _tc_mlp(eparts, nparts, graph_globals, We, Wn, Wg,
                   b.reshape(1, D_OUT))


# speedup vs baseline: 23.7792x; 1.3776x over previous
import functools

import jax
import jax.numpy as jnp
from jax import lax
from jax.experimental import pallas as pl
from jax.experimental.pallas import tpu as pltpu
from jax.experimental.pallas import tpu_sc as plsc

N = 10000
E = 320000
B = 64
D_NODE = 128
D_EDGE = 16
D_OUT = 128
L = 16

NC = 2
NS = 16
NW = NC * NS

NPW = 312
NPW_LAST = N - (NW - 1) * NPW
NVR = D_NODE // L

NT = E // 128
TPW = NT // NW
TREM = NT - TPW * NW
TSUB = 16
NSUB_E = 5
SUB_E = TSUB * 128

_F32Z = functools.partial(jnp.zeros, (L,), jnp.float32)


def _sload(ref, idx):
    return ref[pl.ds(idx, L)][0]


def _lower_bound(ids_ref, lo, hi, val, steps):
    l = jnp.int32(lo)
    h = jnp.int32(hi)
    for _ in range(steps):
        mid = (l + h) // 2
        v = _sload(ids_ref, mid)
        active = l < h
        less = v < val
        l = jnp.where(jnp.logical_and(active, less), mid + 1, l)
        h = jnp.where(jnp.logical_and(active, jnp.logical_not(less)), mid, h)
    return l


def _tree_add(vals):
    while len(vals) > 1:
        vals = [a + b for a, b in zip(vals[::2], vals[1::2])]
    return vals[0]


def _group_loads(ebuf, j):
    tj = j // 8
    cj = (j % 8) * L
    return [ebuf[f // 8, tj, f % 8, pl.ds(cj, L)] for f in range(16)]


def _accum_edges(ebuf, eidb, acc_e, start, end):
    g0 = _sload(eidb, start)
    g1 = _sload(eidb, end - 1)
    iot = lax.iota(jnp.int32, 16)
    z = _F32Z()

    def seg_body(g, s):
        e_ = _lower_bound(eidb, s, end, g + 1, 12)
        hg = s // L
        hs = s - hg * L
        he = jnp.minimum(e_ - hg * L, L)
        mh = jnp.logical_and(iot >= hs, iot < he)
        accs = tuple(jnp.where(mh, v, z) for v in _group_loads(ebuf, hg))

        def blk(j, accs):
            return tuple(a + v for a, v in zip(accs, _group_loads(ebuf, j)))

        tg = e_ // L
        accs = lax.fori_loop(hg + 1, tg, blk, accs)

        tlen = jnp.where(tg > hg, e_ - tg * L, 0)
        mt = iot < tlen
        accs = tuple(a + jnp.where(mt, v, z)
                     for a, v in zip(accs, _group_loads(ebuf, tg)))

        gq = g // 8
        gc = (g % 8) * L
        out_v = acc_e[gq, pl.ds(gc, L)]
        for f in range(16):
            out_v = jnp.where(iot == f, out_v + jnp.sum(accs[f]), out_v)
        acc_e[gq, pl.ds(gc, L)] = out_v
        return e_

    lax.fori_loop(g0, g1 + 1, seg_body, start)


def _accum_nodes(nbuf, nidb, acc_n, rows):
    g0 = _sload(nidb, 0)
    g1 = _sload(nidb, rows - 1)

    def seg_body(g, start):
        end = _lower_bound(nidb, start, rows, g + 1, 9)

        def row(i, accs):
            return tuple(accs[j] + nbuf[i, pl.ds(L * j, L)] for j in range(NVR))

        accs = lax.fori_loop(start, end, row,
                             tuple(_F32Z() for _ in range(NVR)))
        for j in range(NVR):
            sl = pl.ds(L * j, L)
            acc_n[g, sl] = acc_n[g, sl] + accs[j]
        return end

    lax.fori_loop(g0, g1 + 1, seg_body, jnp.int32(0))


def _sc_body(edges_hbm, eids_hbm, nodes_hbm, nids_hbm, eout_hbm, nout_hbm,
             ebuf0, ebuf1, eidb0, eidb1, nbuf, nidb, acc_e, acc_n,
             sem0, sem1, nsem):
    c = lax.axis_index("c")
    s = lax.axis_index("s")
    w = s * NC + c
    ebufs = (ebuf0, ebuf1)
    eidbs = (eidb0, eidb1)
    sems = (sem0, sem1)

    nbase = w * NPW
    nh = (pltpu.async_copy(nodes_hbm.at[pl.ds(nbase, NPW_LAST)], nbuf, nsem),
          pltpu.async_copy(nids_hbm.at[pl.ds(nbase, NPW_LAST)],
                           nidb.at[pl.ds(0, NPW_LAST)], nsem))

    z = _F32Z()

    def zero_row(i, _):
        for j in range(NVR):
            acc_n[i, pl.ds(L * j, L)] = z
        return 0

    lax.fori_loop(0, B, zero_row, 0)

    def zero_erow(i, _):
        for j in range(NVR):
            acc_e[i, pl.ds(L * j, L)] = z
        return 0

    lax.fori_loop(0, 8, zero_erow, 0)

    tstart = TPW * w + jnp.minimum(w, TREM)
    tend = tstart + TPW + jnp.where(w < TREM, 1, 0)

    def start_sub(k):
        sl = k % 2
        tb = jnp.minimum(tstart + k * TSUB, NT - TSUB)
        hs = [pltpu.async_copy(edges_hbm.at[a, pl.ds(tb, TSUB)],
                               ebufs[sl].at[a, pl.ds(0, TSUB)], sems[sl])
              for a in range(2)]
        hs.append(pltpu.async_copy(eids_hbm.at[pl.ds(tb * 128, SUB_E)],
                                   eidbs[sl].at[pl.ds(0, SUB_E)], sems[sl]))
        return tb, hs

    pending = {0: start_sub(0)}
    cur = tstart * 128
    for k in range(NSUB_E):
        sl = k % 2
        tb, hs = pending.pop(k)
        for h in hs:
            h.wait()
        if k + 1 < NSUB_E:
            pending[k + 1] = start_sub(k + 1)
        lo = jnp.maximum(cur, tb * 128)
        hi = jnp.minimum(tend * 128, tb * 128 + SUB_E)
        _accum_edges(ebufs[sl], eidbs[sl], acc_e, lo - tb * 128, hi - tb * 128)
        cur = hi

    for h in nh:
        h.wait()
    rows = jnp.where(w == NW - 1, NPW_LAST, NPW).astype(jnp.int32)
    _accum_nodes(nbuf, nidb, acc_n, rows)

    pltpu.sync_copy(acc_e, eout_hbm.at[pl.ds(w * 8, 8)])
    pltpu.sync_copy(acc_n, nout_hbm.at[pl.ds(w * B, B)])


_sc_seg = functools.partial(
    pl.kernel,
    out_type=[
        jax.ShapeDtypeStruct((NW * 8, 128), jnp.float32),
        jax.ShapeDtypeStruct((NW * B, D_NODE), jnp.float32),
    ],
    mesh=plsc.VectorSubcoreMesh(core_axis_name="c", subcore_axis_name="s"),
    scratch_types=[
        pltpu.VMEM((2, TSUB + 1, 8, 128), jnp.float32),
        pltpu.VMEM((2, TSUB + 1, 8, 128), jnp.float32),
        pltpu.VMEM((SUB_E + L,), jnp.int32),
        pltpu.VMEM((SUB_E + L,), jnp.int32),
        pltpu.VMEM((NPW_LAST, D_NODE), jnp.float32),
        pltpu.VMEM((NPW_LAST + L,), jnp.int32),
        pltpu.VMEM((8, 128), jnp.float32),
        pltpu.VMEM((B, D_NODE), jnp.float32),
        pltpu.SemaphoreType.DMA,
        pltpu.SemaphoreType.DMA,
        pltpu.SemaphoreType.DMA,
    ],
    compiler_params=pltpu.CompilerParams(use_tc_tiling_on_sc=False,
                                         needs_layout_passes=False),
)(_sc_body)


def _tc_body(ep_ref, np_ref, g_ref, w_ref, b_ref, out_ref):
    eagg8 = jnp.sum(ep_ref[...].reshape(NW, 8, 128), axis=0)
    nagg = jnp.sum(np_ref[...].reshape(NW, B, D_NODE), axis=0)
    Wm = w_ref[...]
    We = Wm[:D_EDGE]
    tp = [jnp.dot(eagg8[:, 16 * p:16 * (p + 1)], We,
                  preferred_element_type=jnp.float32) for p in range(8)]
    acc = jnp.stack(tp, axis=1).reshape(B, D_OUT)
    acc = acc + jnp.dot(nagg, Wm[D_EDGE:D_EDGE + D_NODE],
                        preferred_element_type=jnp.float32)
    acc = acc + jnp.dot(g_ref[...], Wm[D_EDGE + D_NODE:],
                        preferred_element_type=jnp.float32)
    out_ref[...] = acc + b_ref[...]


def _tc_mlp(eparts, nparts, graph_globals, W, b2):
    return pl.pallas_call(
        _tc_body,
        out_shape=jax.ShapeDtypeStruct((B, D_OUT), jnp.float32),
    )(eparts, nparts, graph_globals, W, b2)


def kernel(nodes, edges, graph_globals, node_graph_ids, edge_graph_ids, W, b):
    eids = edge_graph_ids.astype(jnp.int32)
    nids = node_graph_ids.astype(jnp.int32)
    edges4 = edges.T.reshape(2, 8, NT, 128).transpose(0, 2, 1, 3)
    eparts, nparts = _sc_seg(edges4, eids, nodes, nids)
    return _tc_mlp(eparts, nparts, graph_globals, W, b.reshape(1, D_OUT))
